# baseline (speedup 1.0000x reference)
import jax
import jax.numpy as jnp
from jax.experimental import pallas as pl

N = 10000
E = 160000
HID = 16
HEADS = 8
SLOPE = 0.2


def _rel_conv(feat_src, feat_dst, edge_index, W, rel_emb, W_rel):
    fs = (feat_src @ W).reshape(-1, HEADS, HID)
    fd = (feat_dst @ W).reshape(-1, HEADS, HID)
    rw = (rel_emb @ W_rel).reshape(HEADS, 2 * HID)
    e_src = (fs * rw[:, :HID][None, :, :]).sum(-1)
    e_dst = (fd * rw[:, HID:][None, :, :]).sum(-1)
    src = edge_index[0]
    dst = edge_index[1]
    e = jax.nn.leaky_relu(e_src[src] + e_dst[dst], SLOPE)
    m = jax.ops.segment_max(e, dst, num_segments=N)
    m = jnp.where(jnp.isfinite(m), m, 0.0)
    ex = jnp.exp(e - m[dst])
    s = jax.ops.segment_sum(ex, dst, num_segments=N)
    a = ex / (s[dst] + 1e-16)
    msg = fs[src] * a[:, :, None]
    out = jax.ops.segment_sum(msg, dst, num_segments=N)
    return jax.nn.relu(out.reshape(N, HEADS * HID))


def _cross(stacked, w):
    x = stacked.reshape(stacked.shape[0], -1, HEADS, HID)
    attn = (x * w[None, None, :, :]).sum(-1, keepdims=True)
    attn = jax.nn.softmax(jax.nn.leaky_relu(attn, SLOPE), axis=0)
    out = (x * attn).sum(0)
    return out.reshape(-1, HEADS * HID)


def _copy_kernel(a_ref, b_ref, oa_ref, ob_ref):
    oa_ref[...] = a_ref[...]
    ob_ref[...] = b_ref[...]


def kernel(feat_rel0, feat_rel1, rel_emb0, rel_emb1, edge_index0, edge_index1,
           W_node, W_rel0, W_rel1, prop_W0, prop_b0, prop_W1, prop_b1,
           res_W, res_b, res_alpha, cross_w0, cross_w1):
    out0 = _rel_conv(feat_rel0, feat_rel0, edge_index0, W_node, rel_emb0, W_rel0)
    out1 = _rel_conv(feat_rel1, feat_rel1, edge_index1, W_node, rel_emb1, W_rel1)
    alpha = jax.nn.sigmoid(res_alpha)
    out0 = out0 * alpha + (feat_rel0 @ res_W.T + res_b) * (1.0 - alpha)
    out1 = out1 * alpha + (feat_rel1 @ res_W.T + res_b) * (1.0 - alpha)
    stacked = jnp.stack([out0, out1], axis=0)
    cross0 = _cross(stacked, cross_w0)
    cross1 = _cross(stacked, cross_w1)
    new_emb0 = rel_emb0 @ prop_W0.T + prop_b0
    new_emb1 = rel_emb1 @ prop_W1.T + prop_b1
    c0, c1 = pl.pallas_call(
        _copy_kernel,
        out_shape=[jax.ShapeDtypeStruct(cross0.shape, cross0.dtype),
                   jax.ShapeDtypeStruct(cross1.shape, cross1.dtype)],
    )(cross0, cross1)
    return (c0, c1, new_emb0, new_emb1)


# speedup vs baseline: 30.4617x; 30.4617x over previous
import functools

import jax
import jax.numpy as jnp
from jax import lax
from jax.experimental import pallas as pl
from jax.experimental.pallas import tpu as pltpu
from jax.experimental.pallas import tpu_sc as plsc

N = 10000
E = 160000
D_IN = 128
HID = 16
HEADS = 8
HH = HEADS * HID
SLOPE = 0.2
K = 8.0
INVK = 1.0 / K

NC = 2
NS = 16
EPT = E // NS
RPT = N // NS

QH = 2
NQ = HEADS // QH
CH = 2000
CH2 = 200

_mesh = lambda: plsc.VectorSubcoreMesh(
    core_axis_name="c", subcore_axis_name="s", num_cores=NC, num_subcores=NS)


def _dense_pre_body(feat0, feat1, emb0, emb1, Wn, Ws0, Wd0, Ws1, Wd1,
                    resW, resb, pW0, pb0, pW1, pb1,
                    fs0, fs1, es0, ed0, es1, ed1, res0, res1, ne0, ne1):
    G = (lax.broadcasted_iota(jnp.int32, (HH, HEADS), 0) // HID ==
         lax.broadcasted_iota(jnp.int32, (HH, HEADS), 1)).astype(jnp.float32)
    feats = (feat0, feat1)
    embs = (emb0, emb1)
    Wss = (Ws0, Ws1)
    Wds = (Wd0, Wd1)
    fss = (fs0, fs1)
    ess = (es0, es1)
    eds = (ed0, ed1)
    ress = (res0, res1)
    for r in range(2):
        feat = feats[r][...]
        fs = jnp.dot(feat, Wn[...], preferred_element_type=jnp.float32)
        fss[r][...] = fs
        emb = embs[r][...]
        rw_s = jnp.dot(emb, Wss[r][...], preferred_element_type=jnp.float32)
        rw_d = jnp.dot(emb, Wds[r][...], preferred_element_type=jnp.float32)
        ess[r][...] = jnp.dot(fs * rw_s, G, preferred_element_type=jnp.float32)
        eds[r][...] = jnp.dot(fs * rw_d, G, preferred_element_type=jnp.float32)
        ress[r][...] = lax.dot_general(
            feat, resW[...], (((1,), (1,)), ((), ())),
            preferred_element_type=jnp.float32) + resb[...]

    @pl.when(pl.program_id(0) == 0)
    def _():
        nes = (ne0, ne1)
        pWs = (pW0, pW1)
        pbs = (pb0, pb1)
        for r in range(2):
            nes[r][...] = lax.dot_general(
                embs[r][...][:, :64], pWs[r][...], (((1,), (1,)), ((), ())),
                preferred_element_type=jnp.float32) + pbs[r][...]


def _dense_pre(feat0, feat1, emb0, emb1, Wn, Ws0, Wd0, Ws1, Wd1,
               resW, resb, pW0, pb0, pW1, pb1):
    B = 1000
    grid = (N // B,)
    blk = lambda shape: pl.BlockSpec(shape, lambda i: (0,) * len(shape))
    nblk = lambda shape: pl.BlockSpec(shape, lambda i: (i,) + (0,) * (len(shape) - 1))
    return pl.pallas_call(
        _dense_pre_body,
        grid=grid,
        in_specs=[
            nblk((B, D_IN)), nblk((B, D_IN)),
            blk((1, 64)), blk((1, 64)),
            blk((D_IN, HH)),
            blk((64, HH)), blk((64, HH)),
            blk((64, HH)), blk((64, HH)),
            blk((HH, D_IN)), blk((1, D_IN)),
            blk((256, 64)), blk((1, 256)),
            blk((256, 64)), blk((1, 256)),
        ],
        out_specs=[
            nblk((B, HH)), nblk((B, HH)),
            nblk((B, HEADS)), nblk((B, HEADS)),
            nblk((B, HEADS)), nblk((B, HEADS)),
            nblk((B, D_IN)), nblk((B, D_IN)),
            blk((1, 256)), blk((1, 256)),
        ],
        out_shape=[
            jax.ShapeDtypeStruct((N, HH), jnp.float32),
            jax.ShapeDtypeStruct((N, HH), jnp.float32),
            jax.ShapeDtypeStruct((N, HEADS), jnp.float32),
            jax.ShapeDtypeStruct((N, HEADS), jnp.float32),
            jax.ShapeDtypeStruct((N, HEADS), jnp.float32),
            jax.ShapeDtypeStruct((N, HEADS), jnp.float32),
            jax.ShapeDtypeStruct((N, D_IN), jnp.float32),
            jax.ShapeDtypeStruct((N, D_IN), jnp.float32),
            jax.ShapeDtypeStruct((1, 256), jnp.float32),
            jax.ShapeDtypeStruct((1, 256), jnp.float32),
        ],
    )(feat0, feat1, emb0, emb1, Wn, Ws0, Wd0, Ws1, Wd1,
      resW, resb, pW0, pb0, pW1, pb1)


def _pass1_body(es_hbm, ed_hbm, src_hbm, dst_hbm, sk_out,
                es_loc, ed_loc, sk_loc, src_v, dst_v):
    c = lax.axis_index("c")
    s = lax.axis_index("s")
    e0 = s * EPT
    tbl = c * (HEADS * N)
    for q in range(NQ):
        pltpu.sync_copy(es_hbm.at[pl.ds(tbl + q * QH * N, QH * N)], es_loc)
        pltpu.sync_copy(ed_hbm.at[pl.ds(tbl + q * QH * N, QH * N)], ed_loc)

        def _zero(i, _):
            sk_loc[pl.ds(i * 16, 16)] = jnp.zeros((16,), jnp.float32)
            return 0
        lax.fori_loop(0, QH * N // 16, _zero, 0)

        def _chunk(chunk, _):
            base = c * E + e0 + chunk * CH
            pltpu.sync_copy(src_hbm.at[pl.ds(base, CH)], src_v)
            pltpu.sync_copy(dst_hbm.at[pl.ds(base, CH)], dst_v)

            def _grp(g, _):
                sv = src_v[pl.ds(g * 16, 16)]
                dv = dst_v[pl.ds(g * 16, 16)]
                for h in range(QH):
                    a = plsc.load_gather(es_loc, [sv + h * N])
                    b = plsc.load_gather(ed_loc, [dv + h * N])
                    t = a + b
                    e = jnp.where(t >= 0.0, t, t * SLOPE)
                    p = jnp.exp(e * INVK)
                    plsc.addupdate_scatter(sk_loc, [dv + h * N], p)
                return 0
            lax.fori_loop(0, CH // 16, _grp, 0)
            return 0
        lax.fori_loop(0, EPT // CH, _chunk, 0)

        out0 = c * (NS * HEADS * N) + s * (HEADS * N) + q * QH * N
        pltpu.sync_copy(sk_loc, sk_out.at[pl.ds(out0, QH * N)])


def _pass1(es_t, ed_t, src_plain, dst_plain):
    f = functools.partial(
        pl.kernel,
        out_type=jax.ShapeDtypeStruct((NC * NS * HEADS * N,), jnp.float32),
        mesh=_mesh(),
        compiler_params=pltpu.CompilerParams(needs_layout_passes=False),
        scratch_types=[
            pltpu.VMEM((QH * N,), jnp.float32),
            pltpu.VMEM((QH * N,), jnp.float32),
            pltpu.VMEM((QH * N,), jnp.float32),
            pltpu.VMEM((CH,), jnp.int32),
            pltpu.VMEM((CH,), jnp.int32),
        ],
    )(_pass1_body)
    return f(es_t, ed_t, src_plain, dst_plain)


def _logc_body(sk_ref, c_ref):
    sk = jnp.sum(sk_ref[...], axis=1)
    c_ref[...] = jnp.where(sk > 0.0, K * jnp.log(sk), 0.0)


def _logc(sk_parts):
    Bn = 16000
    HN = HEADS * N
    return pl.pallas_call(
        _logc_body,
        grid=(HN // Bn,),
        in_specs=[pl.BlockSpec((NC, NS, Bn), lambda i: (0, 0, i))],
        out_specs=pl.BlockSpec((NC, Bn), lambda i: (0, i)),
        out_shape=jax.ShapeDtypeStruct((NC, HN), jnp.float32),
    )(sk_parts)


def _pass2_body(es_hbm, ed_hbm, c_hbm, src_hbm, dst_hbm, s_out, ex_out,
                es_loc, ed_loc, c_loc, s_loc, src_v, dst_v, ex_buf):
    c = lax.axis_index("c")
    s = lax.axis_index("s")
    e0 = s * EPT
    tbl = c * (HEADS * N)
    for q in range(NQ):
        pltpu.sync_copy(es_hbm.at[pl.ds(tbl + q * QH * N, QH * N)], es_loc)
        pltpu.sync_copy(ed_hbm.at[pl.ds(tbl + q * QH * N, QH * N)], ed_loc)
        pltpu.sync_copy(c_hbm.at[pl.ds(tbl + q * QH * N, QH * N)], c_loc)

        def _zero(i, _):
            s_loc[pl.ds(i * 16, 16)] = jnp.zeros((16,), jnp.float32)
            return 0
        lax.fori_loop(0, QH * N // 16, _zero, 0)

        def _chunk(chunk, _):
            base = c * E + e0 + chunk * CH
            pltpu.sync_copy(src_hbm.at[pl.ds(base, CH)], src_v)
            pltpu.sync_copy(dst_hbm.at[pl.ds(base, CH)], dst_v)

            def _grp(g, _):
                sv = src_v[pl.ds(g * 16, 16)]
                dv = dst_v[pl.ds(g * 16, 16)]
                for h in range(QH):
                    a = plsc.load_gather(es_loc, [sv + h * N])
                    b = plsc.load_gather(ed_loc, [dv + h * N])
                    cg = plsc.load_gather(c_loc, [dv + h * N])
                    t = a + b
                    e = jnp.where(t >= 0.0, t, t * SLOPE)
                    ex = jnp.exp(e - cg)
                    ex_buf[pl.ds(h * CH + g * 16, 16)] = ex
                    plsc.addupdate_scatter(s_loc, [dv + h * N], ex)
                return 0
            lax.fori_loop(0, CH // 16, _grp, 0)

            for h in range(QH):
                ho = (q * QH + h) * E
                pltpu.sync_copy(ex_buf.at[pl.ds(h * CH, CH)],
                                ex_out.at[pl.ds(c * (HEADS * E) + ho + e0 + chunk * CH, CH)])
            return 0
        lax.fori_loop(0, EPT // CH, _chunk, 0)

        out0 = c * (NS * HEADS * N) + s * (HEADS * N) + q * QH * N
        pltpu.sync_copy(s_loc, s_out.at[pl.ds(out0, QH * N)])


def _pass2(es_t, ed_t, c_t, src_plain, dst_plain):
    f = functools.partial(
        pl.kernel,
        out_type=[
            jax.ShapeDtypeStruct((NC * NS * HEADS * N,), jnp.float32),
            jax.ShapeDtypeStruct((NC * HEADS * E,), jnp.float32),
        ],
        mesh=_mesh(),
        compiler_params=pltpu.CompilerParams(needs_layout_passes=False),
        scratch_types=[
            pltpu.VMEM((QH * N,), jnp.float32),
            pltpu.VMEM((QH * N,), jnp.float32),
            pltpu.VMEM((QH * N,), jnp.float32),
            pltpu.VMEM((QH * N,), jnp.float32),
            pltpu.VMEM((CH,), jnp.int32),
            pltpu.VMEM((CH,), jnp.int32),
            pltpu.VMEM((QH * CH,), jnp.float32),
        ],
    )(_pass2_body)
    return f(es_t, ed_t, c_t, src_plain, dst_plain)


def _pass3_body(fs_hbm, ex_hbm, src_hbm, dst_hbm, u_out,
                fs_rows, ex_loc, src_v, dst_v, u_acc, sem):
    c = lax.axis_index("c")
    s = lax.axis_index("s")
    e0 = s * EPT

    def _zrow(i, _):
        for kk in range(HH // 16):
            fs_rows[i, pl.ds(kk * 16, 16)] = jnp.zeros((16,), jnp.float32)
        return 0
    lax.fori_loop(0, CH2, _zrow, 0)
    RPT8 = 624
    r0 = s * RPT8
    pltpu.sync_copy(fs_rows.at[pl.ds(0, CH2)], u_acc.at[pl.ds(r0, CH2)])
    pltpu.sync_copy(fs_rows.at[pl.ds(0, CH2)], u_acc.at[pl.ds(r0 + CH2, CH2)])
    pltpu.sync_copy(fs_rows.at[pl.ds(0, CH2)], u_acc.at[pl.ds(r0 + 2 * CH2, CH2)])
    pltpu.sync_copy(fs_rows.at[pl.ds(0, RPT8 - 3 * CH2)],
                    u_acc.at[pl.ds(r0 + 3 * CH2, RPT8 - 3 * CH2)])

    @pl.when(s == NS - 1)
    def _():
        pltpu.sync_copy(fs_rows.at[pl.ds(0, N - NS * RPT8)],
                        u_acc.at[pl.ds(NS * RPT8, N - NS * RPT8)])
    plsc.subcore_barrier()

    def _chunk(chunk, _):
        base = c * E + e0 + chunk * CH2
        pltpu.sync_copy(src_hbm.at[pl.ds(base, CH2)], src_v)
        pltpu.sync_copy(dst_hbm.at[pl.ds(base, CH2)], dst_v)
        pltpu.async_copy(fs_hbm.at[src_v], fs_rows, sem).wait()
        for h in range(HEADS):
            pltpu.sync_copy(
                ex_hbm.at[pl.ds(c * (HEADS * E) + h * E + e0 + chunk * CH2, CH2)],
                ex_loc.at[pl.ds(h * CH2, CH2)])

        def _edge(j, _):
            for h in range(HEADS):
                idx = jnp.full((16,), h * CH2, jnp.int32) + j
                exv = plsc.load_gather(ex_loc, [idx])
                v = fs_rows[j, pl.ds(h * HID, HID)]
                fs_rows[j, pl.ds(h * HID, HID)] = v * exv
            return 0
        lax.fori_loop(0, CH2, _edge, 0)
        pltpu.sync_copy(fs_rows, u_acc.at[dst_v], add=True)
        return 0
    lax.fori_loop(0, EPT // CH2, _chunk, 0)

    plsc.subcore_barrier()
    pltpu.sync_copy(u_acc.at[pl.ds(r0, RPT8)],
                    u_out.at[pl.ds(c * N + r0, RPT8)])

    @pl.when(s == NS - 1)
    def _():
        pltpu.sync_copy(u_acc.at[pl.ds(NS * RPT8, N - NS * RPT8)],
                        u_out.at[pl.ds(c * N + NS * RPT8, N - NS * RPT8)])


def _pass3(fs_all, ex_all, src_off, dst_plain):
    f = functools.partial(
        pl.kernel,
        out_type=jax.ShapeDtypeStruct((NC * N, HH), jnp.float32),
        mesh=_mesh(),
        compiler_params=pltpu.CompilerParams(needs_layout_passes=False),
        scratch_types=[
            pltpu.VMEM((CH2, HH), jnp.float32),
            pltpu.VMEM((HEADS * CH2,), jnp.float32),
            pltpu.VMEM((CH2,), jnp.int32),
            pltpu.VMEM((CH2,), jnp.int32),
            pltpu.VMEM_SHARED((N, HH), jnp.float32),
            pltpu.SemaphoreType.DMA,
        ],
    )(_pass3_body)
    return f(fs_all, ex_all, src_off, dst_plain)


def _final_body(u_ref, sp_ref, res0_ref, res1_ref, alpha_ref, cw0_ref, cw1_ref,
                out0_ref, out1_ref):
    G = (lax.broadcasted_iota(jnp.int32, (HH, HEADS), 0) // HID ==
         lax.broadcasted_iota(jnp.int32, (HH, HEADS), 1)).astype(jnp.float32)
    GT = (lax.broadcasted_iota(jnp.int32, (HEADS, HH), 1) // HID ==
          lax.broadcasted_iota(jnp.int32, (HEADS, HH), 0)).astype(jnp.float32)
    alpha = jax.nn.sigmoid(alpha_ref[0, 0])
    ress = (res0_ref, res1_ref)
    o = []
    for r in range(2):
        s_blk = jnp.sum(sp_ref[r], axis=0)
        s_exp = jnp.dot(s_blk, GT, preferred_element_type=jnp.float32)
        u = u_ref[r]
        outg = jnp.where(s_exp > 0.0, u / s_exp, 0.0)
        outg = jnp.maximum(outg, 0.0)
        o.append(alpha * outg + (1.0 - alpha) * ress[r][...])
    outs = (out0_ref, out1_ref)
    cws = (cw0_ref, cw1_ref)
    for w in range(2):
        cw = cws[w][...]
        l0 = jnp.dot(o[0] * cw, G, preferred_element_type=jnp.float32)
        l1 = jnp.dot(o[1] * cw, G, preferred_element_type=jnp.float32)
        l0 = jnp.where(l0 >= 0.0, l0, l0 * SLOPE)
        l1 = jnp.where(l1 >= 0.0, l1, l1 * SLOPE)
        m = jnp.maximum(l0, l1)
        a0 = jnp.exp(l0 - m)
        a1 = jnp.exp(l1 - m)
        den = a0 + a1
        a0e = jnp.dot(a0 / den, GT, preferred_element_type=jnp.float32)
        a1e = jnp.dot(a1 / den, GT, preferred_element_type=jnp.float32)
        outs[w][...] = a0e * o[0] + a1e * o[1]


def _final(u, sp, res0, res1, alpha, cw0, cw1):
    B = 1000
    return pl.pallas_call(
        _final_body,
        grid=(N // B,),
        in_specs=[
            pl.BlockSpec((2, B, HH), lambda i: (0, i, 0)),
            pl.BlockSpec((2, NS, B, HEADS), lambda i: (0, 0, i, 0)),
            pl.BlockSpec((B, D_IN), lambda i: (i, 0)),
            pl.BlockSpec((B, D_IN), lambda i: (i, 0)),
            pl.BlockSpec((1, 1), lambda i: (0, 0)),
            pl.BlockSpec((1, HH), lambda i: (0, 0)),
            pl.BlockSpec((1, HH), lambda i: (0, 0)),
        ],
        out_specs=[
            pl.BlockSpec((B, HH), lambda i: (i, 0)),
            pl.BlockSpec((B, HH), lambda i: (i, 0)),
        ],
        out_shape=[
            jax.ShapeDtypeStruct((N, HH), jnp.float32),
            jax.ShapeDtypeStruct((N, HH), jnp.float32),
        ],
    )(u, sp, res0, res1, alpha, cw0, cw1)


def kernel(feat_rel0, feat_rel1, rel_emb0, rel_emb1, edge_index0, edge_index1,
           W_node, W_rel0, W_rel1, prop_W0, prop_b0, prop_W1, prop_b1,
           res_W, res_b, res_alpha, cross_w0, cross_w1):
    emb0 = rel_emb0.reshape(1, 64)
    emb1 = rel_emb1.reshape(1, 64)
    Wr0 = W_rel0.reshape(64, HEADS, 2 * HID)
    Wr1 = W_rel1.reshape(64, HEADS, 2 * HID)
    Ws0 = Wr0[:, :, :HID].reshape(64, HH)
    Wd0 = Wr0[:, :, HID:].reshape(64, HH)
    Ws1 = Wr1[:, :, :HID].reshape(64, HH)
    Wd1 = Wr1[:, :, HID:].reshape(64, HH)

    fs0, fs1, es0, ed0, es1, ed1, res0, res1, ne0, ne1 = _dense_pre(
        feat_rel0, feat_rel1, emb0, emb1, W_node, Ws0, Wd0, Ws1, Wd1,
        res_W, res_b.reshape(1, D_IN), prop_W0, prop_b0.reshape(1, 256),
        prop_W1, prop_b1.reshape(1, 256))

    es_t = jnp.concatenate([es0.T.reshape(-1), es1.T.reshape(-1)])
    ed_t = jnp.concatenate([ed0.T.reshape(-1), ed1.T.reshape(-1)])

    e0 = edge_index0.astype(jnp.int32)
    e1 = edge_index1.astype(jnp.int32)
    src_plain = jnp.concatenate([e0[0], e1[0]])
    dst_plain = jnp.concatenate([e0[1], e1[1]])
    src_off = jnp.concatenate([e0[0], e1[0] + N])

    sk_parts = _pass1(es_t, ed_t, src_plain, dst_plain)
    c_t = _logc(sk_parts.reshape(NC, NS, HEADS * N)).reshape(-1)
    s_parts, ex_all = _pass2(es_t, ed_t, c_t, src_plain, dst_plain)

    fs_all = jnp.concatenate([fs0, fs1], axis=0)
    u = _pass3(fs_all, ex_all, src_off, dst_plain)

    cross0, cross1 = _final(
        u.reshape(NC, N, HH),
        s_parts.reshape(NC, NS, HEADS, N).transpose(0, 1, 3, 2),
        res0, res1, res_alpha.reshape(1, 1),
        cross_w0.reshape(1, HH), cross_w1.reshape(1, HH))

    return (cross0, cross1, ne0.reshape(256), ne1.reshape(256))
